# baseline (speedup 1.0000x reference)
import jax
import jax.numpy as jnp
from jax.experimental import pallas as pl
from jax.experimental.pallas import tpu as pltpu


def _round_up(v, m):
    return ((v + m - 1) // m) * m


def _linear_kernel(x_ref, w_ref, b_ref, o_ref):
    acc = jax.lax.dot_general(
        x_ref[...], w_ref[...],
        dimension_numbers=(((1,), (1,)), ((), ())),
        preferred_element_type=jnp.float32,
    )
    o_ref[...] = (acc + b_ref[...]).astype(o_ref.dtype)


def kernel(x, weight, bias):
    B, D = x.shape
    O = weight.shape[0]

    bm = min(2048, _round_up(B, 8))
    Bp = _round_up(B, bm)
    Dp = _round_up(D, 128)
    Op = _round_up(O, 128)

    if (Bp, Dp) != (B, D):
        x = jnp.pad(x, ((0, Bp - B), (0, Dp - D)))
    if (Op, Dp) != (O, D):
        weight = jnp.pad(weight, ((0, Op - O), (0, Dp - D)))
    b2 = bias.reshape(1, O)
    if Op != O:
        b2 = jnp.pad(b2, ((0, 0), (0, Op - O)))

    out = pl.pallas_call(
        _linear_kernel,
        out_shape=jax.ShapeDtypeStruct((Bp, Op), x.dtype),
        grid=(Bp // bm,),
        in_specs=[
            pl.BlockSpec((bm, Dp), lambda i: (i, 0)),
            pl.BlockSpec((Op, Dp), lambda i: (0, 0)),
            pl.BlockSpec((1, Op), lambda i: (0, 0)),
        ],
        out_specs=pl.BlockSpec((bm, Op), lambda i: (i, 0)),
        compiler_params=pltpu.CompilerParams(
            dimension_semantics=("parallel",),
            vmem_limit_bytes=64 * 1024 * 1024,
        ),
    )(x, weight, b2)
    if (Bp, Op) != (B, O):
        out = out[:B, :O]
    return out
